# baseline (speedup 1.0000x reference)
import functools

import jax
import jax.numpy as jnp
from jax import lax
from jax.experimental import pallas as pl
from jax.experimental.pallas import tpu as pltpu
from jax.experimental.pallas import tpu_sc as plsc

BATCH = 4
SEQ = 4096
DIM = 1024

_SC_BATCH = 1
_NC = 2
_NS = 16
_NW = _NC * _NS

_CH_ROWS = 16
_ROWS_PER_W = SEQ // _NW
_NCHUNK = _ROWS_PER_W // _CH_ROWS

_BLK_S = 2048


def _make_sc_add():
    mesh = plsc.VectorSubcoreMesh(core_axis_name="c", subcore_axis_name="s")

    @functools.partial(
        pl.kernel,
        mesh=mesh,
        out_type=jax.ShapeDtypeStruct((_SC_BATCH * SEQ, DIM), jnp.float32),
        compiler_params=pltpu.CompilerParams(use_tc_tiling_on_sc=True),
        scratch_types=[
            pltpu.VMEM((_CH_ROWS, DIM), jnp.float32),
            pltpu.VMEM((_CH_ROWS, DIM), jnp.float32),
            pltpu.VMEM((_CH_ROWS, DIM), jnp.float32),
            pltpu.SemaphoreType.DMA,
            pltpu.SemaphoreType.DMA,
            pltpu.SemaphoreType.DMA,
            pltpu.SemaphoreType.DMA,
        ],
    )
    def sc_add(in_hbm, pos_hbm, out_hbm, pos_v, io0, io1, si0, si1, so0, so1):
        wid = lax.axis_index("s") * _NC + lax.axis_index("c")
        row0 = wid * _ROWS_PER_W

        io = (io0, io1)
        sin = (si0, si1)
        sout = (so0, so1)
        steps = [(ci, b) for ci in range(_NCHUNK) for b in range(_SC_BATCH)]
        nst = len(steps)

        def in_load(t):
            ci, b = steps[t]
            r = b * SEQ + row0 + ci * _CH_ROWS
            return pltpu.async_copy(
                in_hbm.at[pl.ds(r, _CH_ROWS), :], io[t % 2], sin[t % 2]
            )

        load_h = {0: in_load(0)}
        store_h = {}

        for t in range(nst):
            ci, b = steps[t]
            r = b * SEQ + row0 + ci * _CH_ROWS
            buf = t % 2
            if t + 1 < nst:
                if t >= 1:
                    store_h[t - 1].wait()
                load_h[t + 1] = in_load(t + 1)
            if b == 0:
                pltpu.sync_copy(
                    pos_hbm.at[pl.ds(row0 + ci * _CH_ROWS, _CH_ROWS), :], pos_v
                )
            load_h[t].wait()
            io_ref = io[buf]

            def add_row(rr, _):
                @plsc.parallel_loop(0, DIM // 16, unroll=8)
                def add_col(c):
                    s = pl.ds(c * 16, 16)
                    io_ref[rr, s] = io_ref[rr, s] + pos_v[rr, s]

                return 0

            lax.fori_loop(0, _CH_ROWS, add_row, 0)
            store_h[t] = pltpu.async_copy(
                io_ref, out_hbm.at[pl.ds(r, _CH_ROWS), :], sout[buf]
            )

        store_h[nst - 2].wait()
        store_h[nst - 1].wait()

    return sc_add


_sc_add = _make_sc_add()


def _tc_add_kernel(x_ref, p_ref, o_ref):
    o_ref[...] = x_ref[...] + p_ref[...]


def _tc_add(inputs, pos_table):
    nb = BATCH - _SC_BATCH
    return pl.pallas_call(
        _tc_add_kernel,
        grid=(SEQ // _BLK_S, nb),
        in_specs=[
            pl.BlockSpec((1, _BLK_S, DIM), lambda i, b: (b + _SC_BATCH, i, 0)),
            pl.BlockSpec((_BLK_S, DIM), lambda i, b: (i, 0)),
        ],
        out_specs=pl.BlockSpec((1, _BLK_S, DIM), lambda i, b: (b, i, 0)),
        out_shape=jax.ShapeDtypeStruct((nb, SEQ, DIM), jnp.float32),
    )(inputs, pos_table)


def kernel(inputs, pos_table):
    sc_out = _sc_add(inputs.reshape(BATCH * SEQ, DIM), pos_table)
    tc_out = _tc_add(inputs, pos_table)
    return jnp.concatenate([sc_out.reshape(_SC_BATCH, SEQ, DIM), tc_out], axis=0)


# speedup vs baseline: 1.2796x; 1.2796x over previous
import functools

import jax
import jax.numpy as jnp
from jax import lax
from jax.experimental import pallas as pl
from jax.experimental.pallas import tpu as pltpu
from jax.experimental.pallas import tpu_sc as plsc

BATCH = 4
SEQ = 4096
DIM = 1024

_NC = 2
_NS = 16
_NW = _NC * _NS

_CH_ROWS = 16
_ROWS_PER_W = SEQ // _NW
_NCHUNK = _ROWS_PER_W // _CH_ROWS
_NBUF = 3


def _make_sc_add():
    mesh = plsc.VectorSubcoreMesh(core_axis_name="c", subcore_axis_name="s")

    @functools.partial(
        pl.kernel,
        mesh=mesh,
        out_type=jax.ShapeDtypeStruct((BATCH * SEQ, DIM), jnp.float32),
        compiler_params=pltpu.CompilerParams(use_tc_tiling_on_sc=True),
        scratch_types=[
            [pltpu.VMEM((_CH_ROWS, DIM), jnp.float32) for _ in range(2)],
            [pltpu.VMEM((_CH_ROWS, DIM), jnp.float32) for _ in range(_NBUF)],
            [pltpu.SemaphoreType.DMA for _ in range(2)],
            [pltpu.SemaphoreType.DMA for _ in range(_NBUF)],
            [pltpu.SemaphoreType.DMA for _ in range(_NBUF)],
        ],
    )
    def sc_add(in_hbm, pos_hbm, out_hbm, pos_bufs, io, spos, sin, sout):
        wid = lax.axis_index("s") * _NC + lax.axis_index("c")
        row0 = wid * _ROWS_PER_W

        steps = [(ci, b) for ci in range(_NCHUNK) for b in range(BATCH)]
        nst = len(steps)

        def in_load(t):
            ci, b = steps[t]
            r = b * SEQ + row0 + ci * _CH_ROWS
            return pltpu.async_copy(
                in_hbm.at[pl.ds(r, _CH_ROWS), :], io[t % _NBUF], sin[t % _NBUF]
            )

        def pos_load(ci):
            return pltpu.async_copy(
                pos_hbm.at[pl.ds(row0 + ci * _CH_ROWS, _CH_ROWS), :],
                pos_bufs[ci % 2],
                spos[ci % 2],
            )

        pos_h = {0: pos_load(0)}
        load_h = {t: in_load(t) for t in range(min(_NBUF - 1, nst))}
        store_h = {}

        for t in range(nst):
            ci, b = steps[t]
            r = b * SEQ + row0 + ci * _CH_ROWS
            buf = t % _NBUF
            if t + _NBUF - 1 < nst:
                if t >= 1:
                    store_h[t - 1].wait()
                load_h[t + _NBUF - 1] = in_load(t + _NBUF - 1)
            if b == 0:
                pos_h[ci].wait()
                if ci + 1 < _NCHUNK:
                    pos_h[ci + 1] = pos_load(ci + 1)
            pos_ref = pos_bufs[ci % 2]
            load_h[t].wait()
            io_ref = io[buf]

            def add_row(rr, _):
                @plsc.parallel_loop(0, DIM // 16, unroll=8)
                def add_col(c):
                    s = pl.ds(c * 16, 16)
                    io_ref[rr, s] = io_ref[rr, s] + pos_ref[rr, s]

                return 0

            lax.fori_loop(0, _CH_ROWS, add_row, 0)
            store_h[t] = pltpu.async_copy(
                io_ref, out_hbm.at[pl.ds(r, _CH_ROWS), :], sout[buf]
            )

        for t in range(max(0, nst - _NBUF), nst):
            store_h[t].wait()

    return sc_add


_sc_add = _make_sc_add()


def kernel(inputs, pos_table):
    batch, seq, dim = inputs.shape
    out = _sc_add(inputs.reshape(batch * seq, dim), pos_table)
    return out.reshape(batch, seq, dim)


# speedup vs baseline: 1.4601x; 1.1410x over previous
import functools

import jax
import jax.numpy as jnp
from jax import lax
from jax.experimental import pallas as pl
from jax.experimental.pallas import tpu as pltpu
from jax.experimental.pallas import tpu_sc as plsc

BATCH = 4
SEQ = 4096
DIM = 1024

_NC = 2
_NS = 16
_NW = _NC * _NS

_CH_ROWS = 8
_ROWS_PER_W = SEQ // _NW
_NCHUNK = _ROWS_PER_W // _CH_ROWS


def _make_sc_add():
    mesh = plsc.VectorSubcoreMesh(core_axis_name="c", subcore_axis_name="s")

    @functools.partial(
        pl.kernel,
        mesh=mesh,
        out_type=jax.ShapeDtypeStruct((BATCH * SEQ, DIM), jnp.float32),
        compiler_params=pltpu.CompilerParams(use_tc_tiling_on_sc=True),
        scratch_types=[
            [pltpu.VMEM((_CH_ROWS, DIM), jnp.float32) for _ in range(2)],
            [
                [pltpu.VMEM((_CH_ROWS, DIM), jnp.float32) for _ in range(BATCH)]
                for _ in range(2)
            ],
            [pltpu.SemaphoreType.DMA for _ in range(2)],
            [pltpu.SemaphoreType.DMA for _ in range(2)],
            [pltpu.SemaphoreType.DMA for _ in range(2)],
        ],
    )
    def sc_add(in_hbm, pos_hbm, out_hbm, pos_bufs, io, spos, sin, sout):
        wid = lax.axis_index("s") * _NC + lax.axis_index("c")
        row0 = wid * _ROWS_PER_W

        def in_load(ci):
            ring = ci % 2
            return [
                pltpu.async_copy(
                    in_hbm.at[pl.ds(b * SEQ + row0 + ci * _CH_ROWS, _CH_ROWS), :],
                    io[ring][b],
                    sin[ring],
                )
                for b in range(BATCH)
            ]

        def pos_load(ci):
            return pltpu.async_copy(
                pos_hbm.at[pl.ds(row0 + ci * _CH_ROWS, _CH_ROWS), :],
                pos_bufs[ci % 2],
                spos[ci % 2],
            )

        pos_h = {0: pos_load(0)}
        load_h = {0: in_load(0)}
        store_h = {}

        for ci in range(_NCHUNK):
            ring = ci % 2
            if ci + 1 < _NCHUNK:
                if ci >= 1:
                    for h in store_h[ci - 1]:
                        h.wait()
                load_h[ci + 1] = in_load(ci + 1)
                pos_h[ci + 1] = pos_load(ci + 1)
            pos_h[ci].wait()
            for h in load_h[ci]:
                h.wait()
            pr = pos_bufs[ci % 2]
            bufs = io[ring]

            def add_row(rr, _):
                @plsc.parallel_loop(0, DIM // 16, unroll=4)
                def add_col(c):
                    s = pl.ds(c * 16, 16)
                    p = pr[rr, s]
                    for b in range(BATCH):
                        bufs[b][rr, s] = bufs[b][rr, s] + p

                return 0

            lax.fori_loop(0, _CH_ROWS, add_row, 0)
            store_h[ci] = [
                pltpu.async_copy(
                    bufs[b],
                    out_hbm.at[pl.ds(b * SEQ + row0 + ci * _CH_ROWS, _CH_ROWS), :],
                    sout[ring],
                )
                for b in range(BATCH)
            ]

        for ci in range(max(0, _NCHUNK - 2), _NCHUNK):
            for h in store_h[ci]:
                h.wait()

    return sc_add


_sc_add = _make_sc_add()


def kernel(inputs, pos_table):
    batch, seq, dim = inputs.shape
    out = _sc_add(inputs.reshape(batch * seq, dim), pos_table)
    return out.reshape(batch, seq, dim)
